# baseline (speedup 1.0000x reference)
import functools

import jax
import jax.numpy as jnp
from jax import lax
from jax.experimental import pallas as pl
from jax.experimental.pallas import tpu as pltpu
from jax.experimental.pallas import tpu_sc as plsc

VOCAB_ = 1000000
EMBED_ = 64
BATCH_ = 16384

NC_ = 2
NS_ = 16
L_ = 16
NW_ = NC_ * NS_
NK_ = 2 * BATCH_ // NW_
CW_ = 512
TAIL0_ = (VOCAB_ // 128) * 128
RING_ = 8

_LOG1P_COEFS = (
    3.6622423e-03, -2.2628007e-02, 6.5735526e-02, -1.2447195e-01,
    1.8421386e-01, -2.4618968e-01, 3.3278534e-01, -4.9995893e-01,
    9.9999881e-01, 6.0578476e-09,
)


def _make_gather_kernel():
    mesh = plsc.VectorSubcoreMesh(core_axis_name="c", subcore_axis_name="s")

    @functools.partial(
        pl.kernel, mesh=mesh,
        out_type=jax.ShapeDtypeStruct((2 * BATCH_ * EMBED_,), jnp.float32),
        scratch_types=[
            pltpu.VMEM((NK_ + L_,), jnp.int32),
            pltpu.VMEM((NK_ + L_,), jnp.int32),
            pltpu.VMEM((EMBED_, CW_), jnp.float32),
            pltpu.VMEM((EMBED_, CW_), jnp.float32),
            pltpu.VMEM((EMBED_, EMBED_), jnp.float32),
            pltpu.VMEM((RING_ * EMBED_,), jnp.float32),
            pltpu.SemaphoreType.DMA,
            pltpu.SemaphoreType.DMA,
            pltpu.SemaphoreType.DMA,
        ],
        compiler_params=pltpu.CompilerParams(needs_layout_passes=False),
    )
    def k1(skeys_h, sorder_h, tableT_h, tail_h, stage_h,
           keyb, posb, chunk0, chunk1, tailb, ring, semc, semp, semw):
        wid = lax.axis_index("s") * NC_ + lax.axis_index("c")
        pltpu.sync_copy(skeys_h.at[wid], keyb.at[pl.ds(0, NK_)])
        pltpu.sync_copy(sorder_h.at[wid], posb.at[pl.ds(0, NK_)])
        pltpu.sync_copy(tail_h, tailb)

        iot = lax.iota(jnp.int32, L_)

        def emit_row(b2, q, vals4):
            slot = lax.rem(q, RING_)

            @pl.when(q >= RING_)
            def _():
                pltpu.make_async_copy(
                    ring.at[pl.ds(0, EMBED_)],
                    stage_h.at[pl.ds(0, EMBED_)], semw).wait()
            for kk in range(4):
                ring[pl.ds(slot * EMBED_ + kk * L_, L_)] = vals4[kk]
            pltpu.async_copy(
                ring.at[pl.ds(slot * EMBED_, EMBED_)],
                stage_h.at[pl.ds(b2 * EMBED_, EMBED_)], semw)

        KMAX = TAIL0_ // CW_ - 1

        def fetch(k, buf, sem):
            base = k * CW_
            for cb in range(EMBED_ // 8):
                pltpu.async_copy(
                    tableT_h.at[pl.ds(cb * 8, 8), pl.ds(base, CW_)],
                    buf.at[pl.ds(cb * 8, 8), :], sem)

        def wait8(sem):
            for cb in range(EMBED_ // 8):
                pltpu.make_async_copy(
                    tableT_h.at[pl.ds(0, 8), pl.ds(0, CW_)],
                    chunk0.at[pl.ds(0, 8), :], sem).wait()

        def main(g, carry):
            k_cur, kp, q = carry
            kvec = keyb[pl.ds(g * L_, L_)]
            ovec = posb[pl.ds(g * L_, L_)]
            for jj in range(L_):
                key = kvec[jj]
                live = key < TAIL0_
                k_new = jnp.where(live, key >> 9, k_cur)
                change = live & (k_new != k_cur)
                hit = change & (k_new == kp)
                miss = change & (k_new != kp)
                stale = miss & (kp != -2)
                par = k_new & 1

                @pl.when(hit | stale)
                def _():
                    wait8(semp)

                @pl.when(miss & (par == 0))
                def _(k_new=k_new):
                    fetch(k_new, chunk0, semc)
                    wait8(semc)

                @pl.when(miss & (par == 1))
                def _(k_new=k_new):
                    fetch(k_new, chunk1, semc)
                    wait8(semc)

                pf = change & (k_new + 1 <= KMAX)

                @pl.when(pf & (par == 1))
                def _(k_new=k_new):
                    fetch(k_new + 1, chunk0, semp)

                @pl.when(pf & (par == 0))
                def _(k_new=k_new):
                    fetch(k_new + 1, chunk1, semp)

                kp = jnp.where(change, jnp.where(pf, k_new + 1,
                                                 jnp.int32(-2)), kp)

                @pl.when(live & (par == 0))
                def _(key=key, k_new=k_new, jj=jj, q=q):
                    rl = jnp.broadcast_to(key - k_new * CW_, (L_,))
                    vals4 = [plsc.load_gather(chunk0, [kk * L_ + iot, rl])
                             for kk in range(4)]
                    emit_row(ovec[jj], q, vals4)

                @pl.when(live & (par == 1))
                def _(key=key, k_new=k_new, jj=jj, q=q):
                    rl = jnp.broadcast_to(key - k_new * CW_, (L_,))
                    vals4 = [plsc.load_gather(chunk1, [kk * L_ + iot, rl])
                             for kk in range(4)]
                    emit_row(ovec[jj], q, vals4)

                k_cur = k_new
                q = q + jnp.where(live, 1, 0)
            return (k_cur, kp, q)

        k_end, kp_end, q_end = lax.fori_loop(
            0, NK_ // L_, main,
            (jnp.int32(-1), jnp.int32(-2), jnp.int32(0)))

        @pl.when(kp_end != -2)
        def _():
            wait8(semp)

        def tail(g, q):
            kvec = keyb[pl.ds(g * L_, L_)]
            ovec = posb[pl.ds(g * L_, L_)]
            for jj in range(L_):
                key = kvec[jj]
                live = key >= TAIL0_

                @pl.when(live)
                def _(key=key, jj=jj, q=q):
                    rl = jnp.broadcast_to(key - TAIL0_, (L_,))
                    vals4 = [plsc.load_gather(tailb, [rl, kk * L_ + iot])
                             for kk in range(4)]
                    emit_row(ovec[jj], q, vals4)

                q = q + jnp.where(live, 1, 0)
            return q

        q_fin = lax.fori_loop(0, NK_ // L_, tail, q_end)

        def drain(j, carry):
            pltpu.make_async_copy(
                ring.at[pl.ds(0, EMBED_)],
                stage_h.at[pl.ds(0, EMBED_)], semw).wait()
            return carry

        lax.fori_loop(0, RING_, drain, q_fin)

    return k1


def _make_dot_kernel():
    mesh = plsc.VectorSubcoreMesh(core_axis_name="c", subcore_axis_name="s")
    bpw = BATCH_ // NW_

    @functools.partial(
        pl.kernel, mesh=mesh,
        out_type=jax.ShapeDtypeStruct((NW_, bpw), jnp.float32),
        scratch_types=[
            pltpu.VMEM((bpw * EMBED_,), jnp.float32),
            pltpu.VMEM((bpw * EMBED_,), jnp.float32),
            pltpu.VMEM((bpw,), jnp.float32),
        ],
        compiler_params=pltpu.CompilerParams(needs_layout_passes=False),
    )
    def k2(stage_h, out_h, ff, cc, outv):
        wid = lax.axis_index("s") * NC_ + lax.axis_index("c")
        n = bpw * EMBED_
        pltpu.sync_copy(stage_h.at[pl.ds(wid * n, n)], ff)
        pltpu.sync_copy(
            stage_h.at[pl.ds(BATCH_ * EMBED_ + wid * n, n)], cc)

        iot = lax.iota(jnp.int32, L_)

        def group(g, carry):
            flat = (g * L_ + iot) * EMBED_
            acc = jnp.zeros((L_,), jnp.float32)
            for e in range(EMBED_):
                acc = acc + (plsc.load_gather(ff, [flat + e]) *
                             plsc.load_gather(cc, [flat + e]))
            t = jnp.exp(-jnp.abs(acc))
            p = jnp.full((L_,), _LOG1P_COEFS[0], jnp.float32)
            for coef in _LOG1P_COEFS[1:]:
                p = p * t + coef
            outv[pl.ds(g * L_, L_)] = jnp.minimum(acc, 0.0) - p
            return carry

        lax.fori_loop(0, bpw // L_, group, 0)
        pltpu.sync_copy(outv, out_h.at[wid])

    return k2


_K1 = _make_gather_kernel()
_K2 = _make_dot_kernel()


def kernel(focus, context, embeddings):
    keys = jnp.concatenate([focus, context])
    order = jnp.argsort(keys).astype(jnp.int32)
    skeys = jnp.take(keys, order)
    tail = embeddings[TAIL0_:, :]
    stage = _K1(skeys.reshape(NW_, NK_), order.reshape(NW_, NK_),
                embeddings.T, tail)
    out = _K2(stage)
    return out.reshape(BATCH_, 1, 1)


# speedup vs baseline: 1.6827x; 1.6827x over previous
import functools

import jax
import jax.numpy as jnp
from jax import lax
from jax.experimental import pallas as pl
from jax.experimental.pallas import tpu as pltpu
from jax.experimental.pallas import tpu_sc as plsc

VOCAB_ = 1000000
EMBED_ = 64
BATCH_ = 16384

NC_ = 2
NS_ = 16
L_ = 16
NW_ = NC_ * NS_
NK_ = 2 * BATCH_ // NW_
CW_ = 512
TAIL0_ = (VOCAB_ // 128) * 128
RING_ = 8

_LOG1P_COEFS = (
    3.6622423e-03, -2.2628007e-02, 6.5735526e-02, -1.2447195e-01,
    1.8421386e-01, -2.4618968e-01, 3.3278534e-01, -4.9995893e-01,
    9.9999881e-01, 6.0578476e-09,
)


def _make_gather_kernel():
    mesh = plsc.VectorSubcoreMesh(core_axis_name="c", subcore_axis_name="s")

    @functools.partial(
        pl.kernel, mesh=mesh,
        out_type=jax.ShapeDtypeStruct((2 * BATCH_ * EMBED_,), jnp.float32),
        scratch_types=[
            pltpu.VMEM((NK_ + L_,), jnp.int32),
            pltpu.VMEM((NK_ + L_,), jnp.int32),
            pltpu.VMEM((EMBED_, CW_), jnp.float32),
            pltpu.VMEM((EMBED_, EMBED_), jnp.float32),
            pltpu.VMEM((RING_ * EMBED_,), jnp.float32),
            pltpu.SemaphoreType.DMA,
            pltpu.SemaphoreType.DMA,
        ],
        compiler_params=pltpu.CompilerParams(needs_layout_passes=False),
    )
    def k1(skeys_h, sorder_h, tableT_h, tail_h, stage_h,
           keyb, posb, chunk, tailb, ring, semc, semw):
        wid = lax.axis_index("s") * NC_ + lax.axis_index("c")
        pltpu.sync_copy(skeys_h.at[wid], keyb.at[pl.ds(0, NK_)])
        pltpu.sync_copy(sorder_h.at[wid], posb.at[pl.ds(0, NK_)])
        pltpu.sync_copy(tail_h, tailb)

        iot = lax.iota(jnp.int32, L_)

        def emit_row(b2, q, vals4):
            slot = lax.rem(q, RING_)

            @pl.when(q >= RING_)
            def _():
                pltpu.make_async_copy(
                    ring.at[pl.ds(0, EMBED_)],
                    stage_h.at[pl.ds(0, EMBED_)], semw).wait()
            for kk in range(4):
                ring[pl.ds(slot * EMBED_ + kk * L_, L_)] = vals4[kk]
            pltpu.async_copy(
                ring.at[pl.ds(slot * EMBED_, EMBED_)],
                stage_h.at[pl.ds(b2 * EMBED_, EMBED_)], semw)

        def main(g, carry):
            k_cur, q = carry
            kvec = keyb[pl.ds(g * L_, L_)]
            ovec = posb[pl.ds(g * L_, L_)]
            for jj in range(L_):
                key = kvec[jj]
                live = key < TAIL0_
                k_new = jnp.where(live, key >> 9, k_cur)

                @pl.when(live & (k_new != k_cur))
                def _(k_new=k_new):
                    base = k_new * CW_
                    for cb in range(EMBED_ // 8):
                        pltpu.async_copy(
                            tableT_h.at[pl.ds(cb * 8, 8), pl.ds(base, CW_)],
                            chunk.at[pl.ds(cb * 8, 8), :], semc)
                    for cb in range(EMBED_ // 8):
                        pltpu.make_async_copy(
                            tableT_h.at[pl.ds(0, 8), pl.ds(0, CW_)],
                            chunk.at[pl.ds(0, 8), :], semc).wait()

                @pl.when(live)
                def _(key=key, k_new=k_new, jj=jj, q=q):
                    rl = jnp.broadcast_to(key - k_new * CW_, (L_,))
                    vals4 = [plsc.load_gather(chunk, [kk * L_ + iot, rl])
                             for kk in range(4)]
                    emit_row(ovec[jj], q, vals4)

                k_cur = k_new
                q = q + jnp.where(live, 1, 0)
            return (k_cur, q)

        k_end, q_end = lax.fori_loop(
            0, NK_ // L_, main, (jnp.int32(-1), jnp.int32(0)))

        def tail(g, q):
            kvec = keyb[pl.ds(g * L_, L_)]
            ovec = posb[pl.ds(g * L_, L_)]
            for jj in range(L_):
                key = kvec[jj]
                live = key >= TAIL0_

                @pl.when(live)
                def _(key=key, jj=jj, q=q):
                    rl = jnp.broadcast_to(key - TAIL0_, (L_,))
                    vals4 = [plsc.load_gather(tailb, [rl, kk * L_ + iot])
                             for kk in range(4)]
                    emit_row(ovec[jj], q, vals4)

                q = q + jnp.where(live, 1, 0)
            return q

        q_fin = lax.fori_loop(0, NK_ // L_, tail, q_end)

        def drain(j, carry):
            pltpu.make_async_copy(
                ring.at[pl.ds(0, EMBED_)],
                stage_h.at[pl.ds(0, EMBED_)], semw).wait()
            return carry

        lax.fori_loop(0, RING_, drain, q_fin)

    return k1


def _make_dot_kernel():
    mesh = plsc.VectorSubcoreMesh(core_axis_name="c", subcore_axis_name="s")
    bpw = BATCH_ // NW_

    @functools.partial(
        pl.kernel, mesh=mesh,
        out_type=jax.ShapeDtypeStruct((NW_, bpw), jnp.float32),
        scratch_types=[
            pltpu.VMEM((bpw * EMBED_,), jnp.float32),
            pltpu.VMEM((bpw * EMBED_,), jnp.float32),
            pltpu.VMEM((bpw,), jnp.float32),
        ],
        compiler_params=pltpu.CompilerParams(needs_layout_passes=False),
    )
    def k2(stage_h, out_h, ff, cc, outv):
        wid = lax.axis_index("s") * NC_ + lax.axis_index("c")
        n = bpw * EMBED_
        pltpu.sync_copy(stage_h.at[pl.ds(wid * n, n)], ff)
        pltpu.sync_copy(
            stage_h.at[pl.ds(BATCH_ * EMBED_ + wid * n, n)], cc)

        iot = lax.iota(jnp.int32, L_)

        def group(g, carry):
            flat = (g * L_ + iot) * EMBED_
            acc = jnp.zeros((L_,), jnp.float32)
            for e in range(EMBED_):
                acc = acc + (plsc.load_gather(ff, [flat + e]) *
                             plsc.load_gather(cc, [flat + e]))
            t = jnp.exp(-jnp.abs(acc))
            p = jnp.full((L_,), _LOG1P_COEFS[0], jnp.float32)
            for coef in _LOG1P_COEFS[1:]:
                p = p * t + coef
            outv[pl.ds(g * L_, L_)] = jnp.minimum(acc, 0.0) - p
            return carry

        lax.fori_loop(0, bpw // L_, group, 0)
        pltpu.sync_copy(outv, out_h.at[wid])

    return k2


_K1 = _make_gather_kernel()
_K2 = _make_dot_kernel()


def kernel(focus, context, embeddings):
    keys = jnp.concatenate([focus, context])
    order = jnp.argsort(keys).astype(jnp.int32)
    skeys = jnp.take(keys, order)
    tail = embeddings[TAIL0_:, :]
    stage = _K1(skeys.reshape(NW_, NK_), order.reshape(NW_, NK_),
                embeddings.T, tail)
    out = _K2(stage)
    return out.reshape(BATCH_, 1, 1)
